# baseline (speedup 1.0000x reference)
import functools

import jax
import jax.numpy as jnp
from jax import lax
from jax.experimental import pallas as pl
from jax.experimental.pallas import tpu as pltpu
from jax.experimental.pallas import tpu_sc as plsc

_REGION = (21, 6, 7, 8, 9, 10, 13, 14, 15, 16, 19, 20,
           22, 25, 26, 27, 28, 31, 32, 33, 34, 35)
_C_USED = 40
_C_OUT = 22
_L = 16
_IN_SPLIT = ((0, 8), (8, 8), (16, 8), (24, 8), (32, 8))
_OUT_SPLIT = ((0, 8), (8, 8), (16, 6))


def kernel(x):
    B, _, C_in, W = x.shape

    info = plsc.get_sparse_core_info()
    nc, ns = info.num_cores, info.num_subcores
    nw = nc * ns
    bpw = B // nw
    ng = bpw // 2
    nfull = W // _L
    tail = W - _L

    mesh = plsc.VectorSubcoreMesh(core_axis_name="c", subcore_axis_name="s")

    @functools.partial(
        pl.kernel,
        out_type=jax.ShapeDtypeStruct((B, 1, _C_OUT, W), x.dtype),
        mesh=mesh,
        scratch_types=[
            pltpu.VMEM((2, _C_USED, W), jnp.float32),
            pltpu.VMEM((2, _C_OUT, W), jnp.float32),
            [[pltpu.SemaphoreType.DMA] * len(_IN_SPLIT)] * 2,
            [[pltpu.SemaphoreType.DMA] * len(_OUT_SPLIT)] * 2,
        ],
    )
    def gather_region(x_hbm, out_hbm, in_buf, out_buf, in_sems, out_sems):
        wid = lax.axis_index("s") * nc + lax.axis_index("c")
        b0 = wid * bpw

        def in_descs(b, slot):
            return [
                pltpu.make_async_copy(
                    x_hbm.at[b, 0, pl.ds(lo, n), :],
                    in_buf.at[slot, pl.ds(lo, n), :],
                    in_sems[slot][p])
                for p, (lo, n) in enumerate(_IN_SPLIT)
            ]

        def out_descs(b, slot):
            return [
                pltpu.make_async_copy(
                    out_buf.at[slot, pl.ds(lo, n), :],
                    out_hbm.at[b, 0, pl.ds(lo, n), :],
                    out_sems[slot][p])
                for p, (lo, n) in enumerate(_OUT_SPLIT)
            ]

        def permute(slot):
            def chunk(k, carry):
                off = k * _L
                for j, r in enumerate(_REGION):
                    out_buf[slot, j, pl.ds(off, _L)] = (
                        in_buf[slot, r, pl.ds(off, _L)])
                return carry
            lax.fori_loop(0, nfull, chunk, 0)
            for j, r in enumerate(_REGION):
                out_buf[slot, j, pl.ds(tail, _L)] = (
                    in_buf[slot, r, pl.ds(tail, _L)])

        for d in in_descs(b0, 0):
            d.start()
        for d in in_descs(b0 + 1, 1):
            d.start()

        def group(g, carry):
            for slot in (0, 1):
                b = b0 + 2 * g + slot
                for d in in_descs(b, slot):
                    d.wait()

                @pl.when(g > 0)
                def _():
                    for d in out_descs(b - 2, slot):
                        d.wait()

                permute(slot)

                @pl.when(g < ng - 1)
                def _():
                    for d in in_descs(b + 2, slot):
                        d.start()

                for d in out_descs(b, slot):
                    d.start()
            return carry

        lax.fori_loop(0, ng, group, 0)
        for d in out_descs(b0 + bpw - 2, 0):
            d.wait()
        for d in out_descs(b0 + bpw - 1, 1):
            d.wait()

    return gather_region(x)


# speedup vs baseline: 1.0470x; 1.0470x over previous
import functools

import jax
import jax.numpy as jnp
from jax import lax
from jax.experimental import pallas as pl
from jax.experimental.pallas import tpu as pltpu
from jax.experimental.pallas import tpu_sc as plsc

_REGION = (21, 6, 7, 8, 9, 10, 13, 14, 15, 16, 19, 20,
           22, 25, 26, 27, 28, 31, 32, 33, 34, 35)
_C_USED = 40
_C_OUT = 22
_L = 16
_IN_SPLIT = ((0, 8), (8, 8), (16, 8), (24, 8), (32, 8))
_OUT_SPLIT = ((0, 8), (8, 8), (16, 6))


def kernel(x):
    B, _, C_in, W = x.shape

    info = plsc.get_sparse_core_info()
    nc, ns = info.num_cores, info.num_subcores
    nw = nc * ns
    bpw = B // nw
    ng = bpw // 2
    nfull = W // _L
    tail = W - _L

    mesh = plsc.VectorSubcoreMesh(core_axis_name="c", subcore_axis_name="s")

    @functools.partial(
        pl.kernel,
        out_type=jax.ShapeDtypeStruct((B, _C_OUT, W), x.dtype),
        mesh=mesh,
        scratch_types=[
            pltpu.VMEM((2, _C_USED, W), jnp.float32),
            pltpu.VMEM((2, _C_OUT, W), jnp.float32),
            [[pltpu.SemaphoreType.DMA] * len(_IN_SPLIT)] * 2,
            [[pltpu.SemaphoreType.DMA] * len(_OUT_SPLIT)] * 2,
        ],
    )
    def gather_region(x_hbm, out_hbm, in_buf, out_buf, in_sems, out_sems):
        wid = lax.axis_index("s") * nc + lax.axis_index("c")
        b0 = wid * bpw

        def in_descs(b, slot):
            return [
                pltpu.make_async_copy(
                    x_hbm.at[b, 0, pl.ds(lo, n), :],
                    in_buf.at[slot, pl.ds(lo, n), :],
                    in_sems[slot][p])
                for p, (lo, n) in enumerate(_IN_SPLIT)
            ]

        def out_descs(b, slot):
            return [
                pltpu.make_async_copy(
                    out_buf.at[slot, pl.ds(lo, n), :],
                    out_hbm.at[b, pl.ds(lo, n), :],
                    out_sems[slot][p])
                for p, (lo, n) in enumerate(_OUT_SPLIT)
            ]

        def permute(slot):
            def chunk(k, carry):
                off = k * _L
                for j, r in enumerate(_REGION):
                    out_buf[slot, j, pl.ds(off, _L)] = (
                        in_buf[slot, r, pl.ds(off, _L)])
                return carry
            lax.fori_loop(0, nfull, chunk, 0)
            for j, r in enumerate(_REGION):
                out_buf[slot, j, pl.ds(tail, _L)] = (
                    in_buf[slot, r, pl.ds(tail, _L)])

        for d in in_descs(b0, 0):
            d.start()
        for d in in_descs(b0 + 1, 1):
            d.start()

        def group(g, carry):
            for slot in (0, 1):
                b = b0 + 2 * g + slot
                for d in in_descs(b, slot):
                    d.wait()

                @pl.when(g > 0)
                def _():
                    for d in out_descs(b - 2, slot):
                        d.wait()

                permute(slot)

                @pl.when(g < ng - 1)
                def _():
                    for d in in_descs(b + 2, slot):
                        d.start()

                for d in out_descs(b, slot):
                    d.start()
            return carry

        lax.fori_loop(0, ng, group, 0)
        for d in out_descs(b0 + bpw - 2, 0):
            d.wait()
        for d in out_descs(b0 + bpw - 1, 1):
            d.wait()

    out = gather_region(x)
    return out.reshape(B, 1, _C_OUT, W)


# speedup vs baseline: 1.1833x; 1.1301x over previous
import functools

import jax
import jax.numpy as jnp
from jax import lax
from jax.experimental import pallas as pl
from jax.experimental.pallas import tpu as pltpu
from jax.experimental.pallas import tpu_sc as plsc

_REGION = (21, 6, 7, 8, 9, 10, 13, 14, 15, 16, 19, 20,
           22, 25, 26, 27, 28, 31, 32, 33, 34, 35)
_C_USED = 40
_C_OUT = 22
_L = 16
_IN_SPLIT = ((0, 8), (8, 8), (16, 8), (24, 8), (32, 8))
_OUT_SPLIT = ((0, 8), (8, 8), (16, 6))


def kernel(x):
    B, _, C_in, W = x.shape
    x3 = x.reshape(B, C_in, W)

    info = plsc.get_sparse_core_info()
    nc, ns = info.num_cores, info.num_subcores
    nw = nc * ns
    bpw = B // nw
    ng = bpw // 2
    nfull = W // _L
    tail = W - _L

    mesh = plsc.VectorSubcoreMesh(core_axis_name="c", subcore_axis_name="s")

    @functools.partial(
        pl.kernel,
        out_type=jax.ShapeDtypeStruct((B, _C_OUT, W), x.dtype),
        mesh=mesh,
        scratch_types=[
            pltpu.VMEM((2, _C_USED, W), jnp.float32),
            pltpu.VMEM((2, _C_OUT, W), jnp.float32),
            [[pltpu.SemaphoreType.DMA] * len(_IN_SPLIT)] * 2,
            [[pltpu.SemaphoreType.DMA] * len(_OUT_SPLIT)] * 2,
        ],
    )
    def gather_region(x_hbm, out_hbm, in_buf, out_buf, in_sems, out_sems):
        wid = lax.axis_index("s") * nc + lax.axis_index("c")
        b0 = wid * bpw

        def in_descs(b, slot):
            return [
                pltpu.make_async_copy(
                    x_hbm.at[b, pl.ds(lo, n), :],
                    in_buf.at[slot, pl.ds(lo, n), :],
                    in_sems[slot][p])
                for p, (lo, n) in enumerate(_IN_SPLIT)
            ]

        def out_descs(b, slot):
            return [
                pltpu.make_async_copy(
                    out_buf.at[slot, pl.ds(lo, n), :],
                    out_hbm.at[b, pl.ds(lo, n), :],
                    out_sems[slot][p])
                for p, (lo, n) in enumerate(_OUT_SPLIT)
            ]

        def permute(slot):
            def chunk(k, carry):
                off = k * _L
                for j, r in enumerate(_REGION):
                    out_buf[slot, j, pl.ds(off, _L)] = (
                        in_buf[slot, r, pl.ds(off, _L)])
                return carry
            lax.fori_loop(0, nfull, chunk, 0)
            for j, r in enumerate(_REGION):
                out_buf[slot, j, pl.ds(tail, _L)] = (
                    in_buf[slot, r, pl.ds(tail, _L)])

        for d in in_descs(b0, 0):
            d.start()
        for d in in_descs(b0 + 1, 1):
            d.start()

        def group(g, carry):
            for slot in (0, 1):
                b = b0 + 2 * g + slot
                for d in in_descs(b, slot):
                    d.wait()

                @pl.when(g > 0)
                def _():
                    for d in out_descs(b - 2, slot):
                        d.wait()

                permute(slot)

                @pl.when(g < ng - 1)
                def _():
                    for d in in_descs(b + 2, slot):
                        d.start()

                for d in out_descs(b, slot):
                    d.start()
            return carry

        lax.fori_loop(0, ng, group, 0)
        for d in out_descs(b0 + bpw - 2, 0):
            d.wait()
        for d in out_descs(b0 + bpw - 1, 1):
            d.wait()

    out = gather_region(x3)
    return out.reshape(B, 1, _C_OUT, W)
